# baseline (speedup 1.0000x reference)
import functools

import jax
import jax.numpy as jnp
from jax import lax
from jax.experimental import pallas as pl
from jax.experimental.pallas import tpu as pltpu
from jax.experimental.pallas import tpu_sc as plsc

_N = 10000
_E = 320000
_D = 128
_NW = 32
_EPW = _E // _NW
_C = 40
_NCH = _EPW // _C
_NPS = _N // 16
_TN = 1000
_TEB = 2000

_SC_PARAMS = pltpu.CompilerParams(needs_layout_passes=False)


def _dot(a, b, precision=jax.lax.Precision.DEFAULT):
    return jax.lax.dot_general(a, b, (((1,), (0,)), ((), ())),
                               precision=precision,
                               preferred_element_type=jnp.float32)


def _mesh():
    return plsc.VectorSubcoreMesh(core_axis_name="c", subcore_axis_name="s")


def _tca_body(x_ref, p_ref, wa_ref, wb_ref, wf_ref, xa_ref, xb_ref):
    x = x_ref[...]
    pf = _dot(p_ref[...], wf_ref[...])
    xa_ref[...] = _dot(x, wa_ref[...]) + pf
    xb_ref[...] = _dot(x, wb_ref[...]) - pf


def _tca(x, p4, wa, wb, wf4):
    return pl.pallas_call(
        _tca_body,
        grid=(_N // _TN,),
        in_specs=[
            pl.BlockSpec((_TN, _D), lambda t: (t, 0)),
            pl.BlockSpec((_TN, 4), lambda t: (t, 0)),
            pl.BlockSpec((_D, _D), lambda t: (0, 0)),
            pl.BlockSpec((_D, _D), lambda t: (0, 0)),
            pl.BlockSpec((4, _D), lambda t: (0, 0)),
        ],
        out_specs=[
            pl.BlockSpec((_TN, _D), lambda t: (t, 0)),
            pl.BlockSpec((_TN, _D), lambda t: (t, 0)),
        ],
        out_shape=[
            jax.ShapeDtypeStruct((_N, _D), jnp.float32),
            jax.ShapeDtypeStruct((_N, _D), jnp.float32),
        ],
    )(x, p4, wa, wb, wf4)


def _sc1_body(xa_hbm, xb_hbm, pos_hbm, gi2_hbm, gj2_hbm, s_hbm, nsq_hbm,
              idx2_i, idx2_j, ra0, rb0, ra1, rb1, sb0, sb1, nsq0, nsq1,
              pos4v, sga0, sgb0, sga1, sgb1, so0, so1):
    wid = lax.axis_index("s") * 2 + lax.axis_index("c")
    pltpu.sync_copy(pos_hbm, pos4v)
    pltpu.sync_copy(gi2_hbm.at[wid], idx2_i)
    pltpu.sync_copy(gj2_hbm.at[wid], idx2_j)
    ras = (ra0, ra1)
    rbs = (rb0, rb1)
    sbs = (sb0, sb1)
    nsqs = (nsq0, nsq1)
    sgas = (sga0, sga1)
    sgbs = (sgb0, sgb1)
    sos = (so0, so1)

    def start_g(t, b):
        pltpu.async_copy(xa_hbm.at[idx2_i.at[t]], ras[b], sgas[b])
        pltpu.async_copy(xb_hbm.at[idx2_j.at[t]], rbs[b], sgbs[b])

    def process(t, b):
        base = wid * _EPW + t * _C

        @pl.when(t >= 2)
        def _():
            pltpu.make_async_copy(sbs[b], s_hbm.at[pl.ds(base, _C)],
                                  sos[b]).wait()
            pltpu.make_async_copy(nsqs[b], nsq_hbm.at[pl.ds(base, _C)],
                                  sos[b]).wait()

        def grp(k, c2):
            sl = pl.ds(k * 16, 16)
            ji = idx2_i[t, sl] * 3
            jj = idx2_j[t, sl] * 3
            d0 = (plsc.load_gather(pos4v, [ji])
                  - plsc.load_gather(pos4v, [jj]))
            d1 = (plsc.load_gather(pos4v, [ji + 1])
                  - plsc.load_gather(pos4v, [jj + 1]))
            d2 = (plsc.load_gather(pos4v, [ji + 2])
                  - plsc.load_gather(pos4v, [jj + 2]))
            nsqs[b][sl] = d0 * d0 + d1 * d1 + d2 * d2
            return c2

        lax.fori_loop(0, _C // 16, grp, 0)
        pltpu.make_async_copy(xa_hbm.at[idx2_i.at[t]], ras[b],
                              sgas[b]).wait()
        pltpu.make_async_copy(xb_hbm.at[idx2_j.at[t]], rbs[b],
                              sgbs[b]).wait()

        def row(r, c2):
            for cc in range(8):
                sl = pl.ds(cc * 16, 16)
                sbs[b][r, sl] = ras[b][r, sl] + rbs[b][r, sl]
            return c2

        lax.fori_loop(0, _C, row, 0)

        @pl.when(t + 2 < _NCH)
        def _():
            start_g(t + 2, b)

        pltpu.async_copy(sbs[b], s_hbm.at[pl.ds(base, _C)], sos[b])
        pltpu.async_copy(nsqs[b], nsq_hbm.at[pl.ds(base, _C)], sos[b])

    start_g(0, 0)
    start_g(1, 1)

    def pair(p, carry):
        process(2 * p, 0)
        process(2 * p + 1, 1)
        return carry

    lax.fori_loop(0, _NCH // 2, pair, 0)
    base0 = wid * _EPW + (_NCH - 2) * _C
    base1 = wid * _EPW + (_NCH - 1) * _C
    pltpu.make_async_copy(sb0, s_hbm.at[pl.ds(base0, _C)], so0).wait()
    pltpu.make_async_copy(nsq0, nsq_hbm.at[pl.ds(base0, _C)], so0).wait()
    pltpu.make_async_copy(sb1, s_hbm.at[pl.ds(base1, _C)], so1).wait()
    pltpu.make_async_copy(nsq1, nsq_hbm.at[pl.ds(base1, _C)], so1).wait()


def _sc1(xa, xb, pos4, gi2, gj2):
    f = functools.partial(
        pl.kernel,
        mesh=_mesh(),
        compiler_params=_SC_PARAMS,
        out_type=[
            jax.ShapeDtypeStruct((_E, _D), jnp.float32),
            jax.ShapeDtypeStruct((_E,), jnp.float32),
        ],
        scratch_types=[
            pltpu.VMEM((_NCH, _C), jnp.int32),
            pltpu.VMEM((_NCH, _C), jnp.int32),
            pltpu.VMEM((_C, _D), jnp.float32),
            pltpu.VMEM((_C, _D), jnp.float32),
            pltpu.VMEM((_C, _D), jnp.float32),
            pltpu.VMEM((_C, _D), jnp.float32),
            pltpu.VMEM((_C, _D), jnp.float32),
            pltpu.VMEM((_C, _D), jnp.float32),
            pltpu.VMEM((_C,), jnp.float32),
            pltpu.VMEM((_C,), jnp.float32),
            pltpu.VMEM((3 * _N,), jnp.float32),
            pltpu.SemaphoreType.DMA,
            pltpu.SemaphoreType.DMA,
            pltpu.SemaphoreType.DMA,
            pltpu.SemaphoreType.DMA,
            pltpu.SemaphoreType.DMA,
            pltpu.SemaphoreType.DMA,
        ],
    )(_sc1_body)
    return f(xa, xb, pos4, gi2, gj2)


def _tcb_body(s_ref, nsq_ref, wnrm_ref, b0_ref, w1_ref, b1_ref,
              we0_ref, be0_ref, we1_ref, be1_ref, m_ref, wexp_ref):
    nsq = nsq_ref[...]
    nrm = nsq * lax.rsqrt(nsq + 1e-30)
    h = s_ref[...] + nrm * wnrm_ref[...] + b0_ref[...]
    h = jnp.maximum(h, 0.0)
    ee = _dot(h, w1_ref[...]) + b1_ref[...]
    mu = jnp.mean(ee, axis=-1, keepdims=True)
    var = jnp.mean((ee - mu) * (ee - mu), axis=-1, keepdims=True)
    ee = (ee - mu) * lax.rsqrt(var + 1e-5)
    z = jnp.maximum(_dot(ee, we0_ref[...]) + be0_ref[...], 0.0)
    w = jnp.sum(z * we1_ref[...], axis=-1, keepdims=True) + be1_ref[...]
    wexp = jnp.exp(w)
    m_ref[...] = (ee + 1.0) * wexp
    wexp_ref[...] = wexp


def _tcb(s, nsq, wnrm, b0, w1, b1, we0, be0, we1row, be1):
    full = lambda a, b: pl.BlockSpec((a, b), lambda t: (0, 0))
    return pl.pallas_call(
        _tcb_body,
        grid=(_E // _TEB,),
        in_specs=[
            pl.BlockSpec((_TEB, _D), lambda t: (t, 0)),
            pl.BlockSpec((_TEB, 1), lambda t: (t, 0)),
            full(1, _D), full(1, _D),
            full(_D, _D), full(1, _D),
            full(_D, _D), full(1, _D), full(1, _D), full(1, 1),
        ],
        out_specs=[
            pl.BlockSpec((_TEB, _D), lambda t: (t, 0)),
            pl.BlockSpec((_TEB, 1), lambda t: (t, 0)),
        ],
        out_shape=[
            jax.ShapeDtypeStruct((_E, _D), jnp.float32),
            jax.ShapeDtypeStruct((_E, 1), jnp.float32),
        ],
    )(s, nsq, wnrm, b0, w1, b1, we0, be0, we1row, be1)


def _sc2_body(m_hbm, gj2_hbm, aggr_hbm, idx2_j, mbuf0, mbuf1, zbuf,
              sm0, sm1, aggr_sh):
    cid = lax.axis_index("c")
    sid = lax.axis_index("s")
    wid = sid * 2 + cid

    def zrow(r, c2):
        for cc in range(8):
            zbuf[r, pl.ds(cc * 16, 16)] = jnp.zeros((16,), jnp.float32)
        return c2

    lax.fori_loop(0, 16, zrow, 0)

    def zcp(q, c2):
        pltpu.sync_copy(zbuf, aggr_sh.at[pl.ds(sid * 624 + q * 16, 16)])
        return c2

    lax.fori_loop(0, 40, zcp, 0)
    plsc.subcore_barrier()

    pltpu.sync_copy(gj2_hbm.at[wid], idx2_j)
    mbufs = (mbuf0, mbuf1)
    sms = (sm0, sm1)

    def start_m(t, b):
        base = wid * _EPW + t * _C
        pltpu.async_copy(m_hbm.at[pl.ds(base, _C)], mbufs[b], sms[b])

    def process(t, b):
        base = wid * _EPW + t * _C
        pltpu.make_async_copy(m_hbm.at[pl.ds(base, _C)], mbufs[b],
                              sms[b]).wait()
        pltpu.sync_copy(mbufs[b], aggr_sh.at[idx2_j.at[t]], add=True)

        @pl.when(t + 2 < _NCH)
        def _():
            start_m(t + 2, b)

    start_m(0, 0)
    start_m(1, 1)

    def pair(p, carry):
        process(2 * p, 0)
        process(2 * p + 1, 1)
        return carry

    lax.fori_loop(0, _NCH // 2, pair, 0)
    plsc.subcore_barrier()

    pltpu.sync_copy(aggr_sh.at[pl.ds(sid * 624, 640)],
                    aggr_hbm.at[pl.ds(cid * _N + sid * 624, 640)])


def _sc2(m, gj2):
    f = functools.partial(
        pl.kernel,
        mesh=_mesh(),
        compiler_params=_SC_PARAMS,
        out_type=[jax.ShapeDtypeStruct((2 * _N, _D), jnp.float32)],
        scratch_types=[
            pltpu.VMEM((_NCH, _C), jnp.int32),
            pltpu.VMEM((_C, _D), jnp.float32),
            pltpu.VMEM((_C, _D), jnp.float32),
            pltpu.VMEM((16, _D), jnp.float32),
            pltpu.SemaphoreType.DMA,
            pltpu.SemaphoreType.DMA,
            pltpu.VMEM_SHARED((_N, _D), jnp.float32),
        ],
    )(_sc2_body)
    return f(m, gj2)


def _tcc_body(x_ref, a0_ref, a1_ref, wg0a_ref, wg0b_ref, bg0_ref, wg1_ref,
              bg1_ref, wn0a_ref, wn0b_ref, bn0_ref, wn1_ref, bn1_ref,
              gn_ref, tinv_ref, out_ref, seg_ref, acc_ref, yh_ref):
    t = pl.program_id(0)
    a = a0_ref[...] + a1_ref[...]
    seg = jnp.sum(a, axis=-1, keepdims=True) * (1.0 / _D)
    safe = jnp.where(seg > 0.0, seg, 1.0)
    aggr = (a - seg) / safe
    x = x_ref[...]

    hg = jnp.maximum(_dot(x, wg0a_ref[...]) + _dot(aggr, wg0b_ref[...])
                     + bg0_ref[...], 0.0)
    lg = _dot(hg, wg1_ref[...]) + bg1_ref[...]
    part = jnp.sum(lg, axis=0, keepdims=True)

    @pl.when(t == 0)
    def _():
        acc_ref[...] = part

    @pl.when(t > 0)
    def _():
        acc_ref[...] = acc_ref[...] + part

    hn = jnp.maximum(_dot(x, wn0a_ref[...]) + _dot(aggr, wn0b_ref[...])
                     + bn0_ref[...], 0.0)
    o = _dot(hn, wn1_ref[...]) + bn1_ref[...]
    mu = jnp.mean(o, axis=-1, keepdims=True)
    var = jnp.mean((o - mu) * (o - mu), axis=-1, keepdims=True)
    out_ref[...] = (o - mu) * lax.rsqrt(var + 1e-5) + x
    seg_ref[...] = seg

    @pl.when(t == (_N // _TN) - 1)
    def _():
        y = (acc_ref[...] * (1.0 / _N) + gn_ref[...]) * tinv_ref[...]
        mx = jnp.max(y, axis=-1, keepdims=True)
        e = jnp.exp(y - mx)
        ys = e / jnp.sum(e, axis=-1, keepdims=True)
        am = jnp.argmax(ys, axis=-1)[:, None]
        io = jax.lax.broadcasted_iota(jnp.int32, (1, 2), 1)
        oh = (io == am).astype(jnp.float32)
        yh_ref[...] = (oh - ys) + ys


def _tcc(x, a, wg0a, wg0b, bg0, wg1, bg1, wn0a, wn0b, bn0, wn1, bn1,
         gn, tinv):
    full = lambda a_, b_: pl.BlockSpec((a_, b_), lambda t: (0, 0))
    nt = _N // _TN
    return pl.pallas_call(
        _tcc_body,
        grid=(nt,),
        in_specs=[
            pl.BlockSpec((_TN, _D), lambda t: (t, 0)),
            pl.BlockSpec((_TN, _D), lambda t: (t, 0)),
            pl.BlockSpec((_TN, _D), lambda t: (t + nt, 0)),
            full(_D, _D), full(_D, _D), full(1, _D),
            full(_D, 2), full(1, 2),
            full(_D, _D), full(_D, _D), full(1, _D),
            full(_D, _D), full(1, _D),
            full(1, 2), full(1, 1),
        ],
        out_specs=[
            pl.BlockSpec((_TN, _D), lambda t: (t, 0)),
            pl.BlockSpec((_TN, 1), lambda t: (t, 0)),
            pl.BlockSpec((1, 2), lambda t: (0, 0)),
            pl.BlockSpec((1, 2), lambda t: (0, 0)),
        ],
        out_shape=[
            jax.ShapeDtypeStruct((_N, _D), jnp.float32),
            jax.ShapeDtypeStruct((_N, 1), jnp.float32),
            jax.ShapeDtypeStruct((1, 2), jnp.float32),
            jax.ShapeDtypeStruct((1, 2), jnp.float32),
        ],
    )(x, a, a, wg0a, wg0b, bg0, wg1, bg1, wn0a, wn0b, bn0, wn1, bn1,
      gn, tinv)


def _sc3_body(wexp_hbm, gj_hbm, seg_hbm, wn_hbm, segv, idx_j, wbuf):
    wid = lax.axis_index("s") * 2 + lax.axis_index("c")
    base = wid * _EPW
    pltpu.sync_copy(seg_hbm, segv)
    pltpu.sync_copy(gj_hbm.at[pl.ds(base, _EPW)], idx_j)
    pltpu.sync_copy(wexp_hbm.at[pl.ds(base, _EPW)], wbuf)

    def grp(k, c2):
        sl = pl.ds(k * 16, 16)
        sv = plsc.load_gather(segv, [idx_j[sl]])
        wbuf[sl] = wbuf[sl] / sv
        return c2

    lax.fori_loop(0, _EPW // 16, grp, 0)
    pltpu.sync_copy(wbuf, wn_hbm.at[pl.ds(base, _EPW)])


def _sc3(wexp, gj, seg):
    f = functools.partial(
        pl.kernel,
        mesh=_mesh(),
        compiler_params=_SC_PARAMS,
        out_type=[jax.ShapeDtypeStruct((_E,), jnp.float32)],
        scratch_types=[
            pltpu.VMEM((_N,), jnp.float32),
            pltpu.VMEM((_EPW,), jnp.int32),
            pltpu.VMEM((_EPW,), jnp.float32),
        ],
    )(_sc3_body)
    return f(wexp, gj, seg)


def kernel(x, g, pos, temp, W_ei0, b_ei0, W_ei1, b_ei1, W_ew0, b_ew0,
           W_ew1, b_ew1, W_g0, b_g0, W_g1, b_g1, W_nd0, b_nd0, W_nd1,
           b_nd1):
    gi = g[0].astype(jnp.int32)
    gj = g[1].astype(jnp.int32)
    gi2 = gi.reshape(_NW, _NCH, _C)
    gj2 = gj.reshape(_NW, _NCH, _C)
    p4 = jnp.pad(pos.astype(jnp.float32), ((0, 0), (0, 1)))
    pos3 = pos.astype(jnp.float32).reshape(3 * _N)

    wa = W_ei0[4:4 + _D]
    wb = W_ei0[4 + _D:4 + 2 * _D]
    wf4 = jnp.pad(W_ei0[0:3], ((0, 1), (0, 0)))
    wnrm = W_ei0[3:4]
    b0 = b_ei0.reshape(1, _D)
    b1 = b_ei1.reshape(1, _D)
    be0 = b_ew0.reshape(1, _D)
    we1row = W_ew1.reshape(1, _D)
    be1 = b_ew1.reshape(1, 1)
    wg0a = W_g0[:_D]
    wg0b = W_g0[_D:]
    bg0 = b_g0.reshape(1, _D)
    bg1 = b_g1.reshape(1, 2)
    wn0a = W_nd0[:_D]
    wn0b = W_nd0[_D:]
    bn0 = b_nd0.reshape(1, _D)
    bn1 = b_nd1.reshape(1, _D)

    u = jax.random.uniform(jax.random.key(42), (2,), minval=1e-10,
                           maxval=1.0)
    gn = (-jnp.log(-jnp.log(u))).reshape(1, 2)
    tinv = (1.0 / jnp.asarray(temp, jnp.float32)).reshape(1, 1)

    xa, xb = _tca(x, p4, wa, wb, wf4)
    s, nsq = _sc1(xa, xb, pos3, gi2, gj2)
    m, wexp = _tcb(s, nsq.reshape(_E, 1), wnrm, b0, W_ei1, b1, W_ew0,
                   be0, we1row, be1)
    (a,) = _sc2(m, gj2)
    out, seg, _acc, yh = _tcc(x, a, wg0a, wg0b, bg0, W_g1, bg1, wn0a,
                              wn0b, bn0, W_nd1, bn1, gn, tinv)
    (wn,) = _sc3(wexp.reshape(_E), gj, seg.reshape(_N))
    return out, wn.reshape(_E, 1), yh.reshape(2)


# speedup vs baseline: 1.0696x; 1.0696x over previous
import functools

import jax
import jax.numpy as jnp
from jax import lax
from jax.experimental import pallas as pl
from jax.experimental.pallas import tpu as pltpu
from jax.experimental.pallas import tpu_sc as plsc

_N = 10000
_E = 320000
_D = 128
_NW = 32
_EPW = _E // _NW
_C = 40
_NCH = _EPW // _C
_NPS = _N // 16
_TN = 1000
_TEB = 4000

_SC_PARAMS = pltpu.CompilerParams(needs_layout_passes=False)


def _dot(a, b, precision=jax.lax.Precision.DEFAULT):
    return jax.lax.dot_general(a, b, (((1,), (0,)), ((), ())),
                               precision=precision,
                               preferred_element_type=jnp.float32)


def _mesh():
    return plsc.VectorSubcoreMesh(core_axis_name="c", subcore_axis_name="s")


def _tca_body(x_ref, p_ref, wa_ref, wb_ref, wf_ref, xa_ref, xb_ref):
    x = x_ref[...]
    pf = _dot(p_ref[...], wf_ref[...])
    xa_ref[...] = _dot(x, wa_ref[...]) + pf
    xb_ref[...] = _dot(x, wb_ref[...]) - pf


def _tca(x, p4, wa, wb, wf4):
    return pl.pallas_call(
        _tca_body,
        grid=(_N // _TN,),
        in_specs=[
            pl.BlockSpec((_TN, _D), lambda t: (t, 0)),
            pl.BlockSpec((_TN, 4), lambda t: (t, 0)),
            pl.BlockSpec((_D, _D), lambda t: (0, 0)),
            pl.BlockSpec((_D, _D), lambda t: (0, 0)),
            pl.BlockSpec((4, _D), lambda t: (0, 0)),
        ],
        out_specs=[
            pl.BlockSpec((_TN, _D), lambda t: (t, 0)),
            pl.BlockSpec((_TN, _D), lambda t: (t, 0)),
        ],
        out_shape=[
            jax.ShapeDtypeStruct((_N, _D), jnp.float32),
            jax.ShapeDtypeStruct((_N, _D), jnp.float32),
        ],
    )(x, p4, wa, wb, wf4)


def _sc1_body(xa_hbm, xb_hbm, pos_hbm, gi2_hbm, gj2_hbm, s_hbm, nsq_hbm,
              idx2_i, idx2_j, ra0, rb0, ra1, rb1, sb0, sb1, nsq0, nsq1,
              pos4v, sga0, sgb0, sga1, sgb1, so0, so1):
    wid = lax.axis_index("s") * 2 + lax.axis_index("c")
    pltpu.sync_copy(pos_hbm, pos4v)
    pltpu.sync_copy(gi2_hbm.at[wid], idx2_i)
    pltpu.sync_copy(gj2_hbm.at[wid], idx2_j)
    ras = (ra0, ra1)
    rbs = (rb0, rb1)
    sbs = (sb0, sb1)
    nsqs = (nsq0, nsq1)
    sgas = (sga0, sga1)
    sgbs = (sgb0, sgb1)
    sos = (so0, so1)

    def start_g(t, b):
        pltpu.async_copy(xa_hbm.at[idx2_i.at[t]], ras[b], sgas[b])
        pltpu.async_copy(xb_hbm.at[idx2_j.at[t]], rbs[b], sgbs[b])

    def process(t, b):
        base = wid * _EPW + t * _C

        @pl.when(t >= 2)
        def _():
            pltpu.make_async_copy(sbs[b], s_hbm.at[pl.ds(base, _C)],
                                  sos[b]).wait()
            pltpu.make_async_copy(nsqs[b], nsq_hbm.at[pl.ds(base, _C)],
                                  sos[b]).wait()

        def grp(k, c2):
            sl = pl.ds(k * 16, 16)
            ji = idx2_i[t, sl] * 3
            jj = idx2_j[t, sl] * 3
            d0 = (plsc.load_gather(pos4v, [ji])
                  - plsc.load_gather(pos4v, [jj]))
            d1 = (plsc.load_gather(pos4v, [ji + 1])
                  - plsc.load_gather(pos4v, [jj + 1]))
            d2 = (plsc.load_gather(pos4v, [ji + 2])
                  - plsc.load_gather(pos4v, [jj + 2]))
            nsqs[b][sl] = d0 * d0 + d1 * d1 + d2 * d2
            return c2

        lax.fori_loop(0, _C // 16, grp, 0)
        pltpu.make_async_copy(xa_hbm.at[idx2_i.at[t]], ras[b],
                              sgas[b]).wait()
        pltpu.make_async_copy(xb_hbm.at[idx2_j.at[t]], rbs[b],
                              sgbs[b]).wait()

        def row(r, c2):
            for cc in range(8):
                sl = pl.ds(cc * 16, 16)
                sbs[b][r, sl] = ras[b][r, sl] + rbs[b][r, sl]
            return c2

        lax.fori_loop(0, _C, row, 0)

        @pl.when(t + 2 < _NCH)
        def _():
            start_g(t + 2, b)

        pltpu.async_copy(sbs[b], s_hbm.at[pl.ds(base, _C)], sos[b])
        pltpu.async_copy(nsqs[b], nsq_hbm.at[pl.ds(base, _C)], sos[b])

    start_g(0, 0)
    start_g(1, 1)

    def pair(p, carry):
        process(2 * p, 0)
        process(2 * p + 1, 1)
        return carry

    lax.fori_loop(0, _NCH // 2, pair, 0)
    base0 = wid * _EPW + (_NCH - 2) * _C
    base1 = wid * _EPW + (_NCH - 1) * _C
    pltpu.make_async_copy(sb0, s_hbm.at[pl.ds(base0, _C)], so0).wait()
    pltpu.make_async_copy(nsq0, nsq_hbm.at[pl.ds(base0, _C)], so0).wait()
    pltpu.make_async_copy(sb1, s_hbm.at[pl.ds(base1, _C)], so1).wait()
    pltpu.make_async_copy(nsq1, nsq_hbm.at[pl.ds(base1, _C)], so1).wait()


def _sc1(xa, xb, pos4, gi2, gj2):
    f = functools.partial(
        pl.kernel,
        mesh=_mesh(),
        compiler_params=_SC_PARAMS,
        out_type=[
            jax.ShapeDtypeStruct((_E, _D), jnp.float32),
            jax.ShapeDtypeStruct((_E,), jnp.float32),
        ],
        scratch_types=[
            pltpu.VMEM((_NCH, _C), jnp.int32),
            pltpu.VMEM((_NCH, _C), jnp.int32),
            pltpu.VMEM((_C, _D), jnp.float32),
            pltpu.VMEM((_C, _D), jnp.float32),
            pltpu.VMEM((_C, _D), jnp.float32),
            pltpu.VMEM((_C, _D), jnp.float32),
            pltpu.VMEM((_C, _D), jnp.float32),
            pltpu.VMEM((_C, _D), jnp.float32),
            pltpu.VMEM((_C,), jnp.float32),
            pltpu.VMEM((_C,), jnp.float32),
            pltpu.VMEM((3 * _N,), jnp.float32),
            pltpu.SemaphoreType.DMA,
            pltpu.SemaphoreType.DMA,
            pltpu.SemaphoreType.DMA,
            pltpu.SemaphoreType.DMA,
            pltpu.SemaphoreType.DMA,
            pltpu.SemaphoreType.DMA,
        ],
    )(_sc1_body)
    return f(xa, xb, pos4, gi2, gj2)


def _tcb_body(s_ref, nsq_ref, wnrm_ref, b0_ref, w1_ref, b1_ref,
              we0_ref, be0_ref, we1_ref, be1_ref, m_ref, wexp_ref):
    nsq = nsq_ref[...]
    nrm = nsq * lax.rsqrt(nsq + 1e-30)
    h = s_ref[...] + nrm * wnrm_ref[...] + b0_ref[...]
    h = jnp.maximum(h, 0.0)
    ee = _dot(h, w1_ref[...]) + b1_ref[...]
    mu = jnp.mean(ee, axis=-1, keepdims=True)
    var = jnp.mean((ee - mu) * (ee - mu), axis=-1, keepdims=True)
    ee = (ee - mu) * lax.rsqrt(var + 1e-5)
    z = jnp.maximum(_dot(ee, we0_ref[...]) + be0_ref[...], 0.0)
    w = jnp.sum(z * we1_ref[...], axis=-1, keepdims=True) + be1_ref[...]
    wexp = jnp.exp(w)
    m_ref[...] = (ee + 1.0) * wexp
    wexp_ref[...] = wexp


def _tcb(s, nsq, wnrm, b0, w1, b1, we0, be0, we1row, be1):
    full = lambda a, b: pl.BlockSpec((a, b), lambda t: (0, 0))
    return pl.pallas_call(
        _tcb_body,
        grid=(_E // _TEB,),
        in_specs=[
            pl.BlockSpec((_TEB, _D), lambda t: (t, 0)),
            pl.BlockSpec((_TEB, 1), lambda t: (t, 0)),
            full(1, _D), full(1, _D),
            full(_D, _D), full(1, _D),
            full(_D, _D), full(1, _D), full(1, _D), full(1, 1),
        ],
        out_specs=[
            pl.BlockSpec((_TEB, _D), lambda t: (t, 0)),
            pl.BlockSpec((_TEB, 1), lambda t: (t, 0)),
        ],
        out_shape=[
            jax.ShapeDtypeStruct((_E, _D), jnp.float32),
            jax.ShapeDtypeStruct((_E, 1), jnp.float32),
        ],
    )(s, nsq, wnrm, b0, w1, b1, we0, be0, we1row, be1)


def _sc2_body(m_hbm, gj2_hbm, aggr_hbm, idx2_j, mbuf0, mbuf1, zbuf,
              sm0, sm1, aggr_sh):
    cid = lax.axis_index("c")
    sid = lax.axis_index("s")
    wid = sid * 2 + cid

    def zrow(r, c2):
        for cc in range(8):
            zbuf[r, pl.ds(cc * 16, 16)] = jnp.zeros((16,), jnp.float32)
        return c2

    lax.fori_loop(0, 16, zrow, 0)

    def zcp(q, c2):
        pltpu.sync_copy(zbuf, aggr_sh.at[pl.ds(sid * 624 + q * 16, 16)])
        return c2

    lax.fori_loop(0, 40, zcp, 0)
    plsc.subcore_barrier()

    pltpu.sync_copy(gj2_hbm.at[wid], idx2_j)
    mbufs = (mbuf0, mbuf1)
    sms = (sm0, sm1)

    def start_m(t, b):
        base = wid * _EPW + t * _C
        pltpu.async_copy(m_hbm.at[pl.ds(base, _C)], mbufs[b], sms[b])

    def process(t, b):
        base = wid * _EPW + t * _C
        pltpu.make_async_copy(m_hbm.at[pl.ds(base, _C)], mbufs[b],
                              sms[b]).wait()
        pltpu.sync_copy(mbufs[b], aggr_sh.at[idx2_j.at[t]], add=True)

        @pl.when(t + 2 < _NCH)
        def _():
            start_m(t + 2, b)

    start_m(0, 0)
    start_m(1, 1)

    def pair(p, carry):
        process(2 * p, 0)
        process(2 * p + 1, 1)
        return carry

    lax.fori_loop(0, _NCH // 2, pair, 0)
    plsc.subcore_barrier()

    pltpu.sync_copy(aggr_sh.at[pl.ds(sid * 624, 640)],
                    aggr_hbm.at[pl.ds(cid * _N + sid * 624, 640)])


def _sc2(m, gj2):
    f = functools.partial(
        pl.kernel,
        mesh=_mesh(),
        compiler_params=_SC_PARAMS,
        out_type=[jax.ShapeDtypeStruct((2 * _N, _D), jnp.float32)],
        scratch_types=[
            pltpu.VMEM((_NCH, _C), jnp.int32),
            pltpu.VMEM((_C, _D), jnp.float32),
            pltpu.VMEM((_C, _D), jnp.float32),
            pltpu.VMEM((16, _D), jnp.float32),
            pltpu.SemaphoreType.DMA,
            pltpu.SemaphoreType.DMA,
            pltpu.VMEM_SHARED((_N, _D), jnp.float32),
        ],
    )(_sc2_body)
    return f(m, gj2)


def _tcc_body(x_ref, a0_ref, a1_ref, wg0a_ref, wg0b_ref, bg0_ref, wg1_ref,
              bg1_ref, wn0a_ref, wn0b_ref, bn0_ref, wn1_ref, bn1_ref,
              gn_ref, tinv_ref, out_ref, seg_ref, acc_ref, yh_ref):
    t = pl.program_id(0)
    a = a0_ref[...] + a1_ref[...]
    seg = jnp.sum(a, axis=-1, keepdims=True) * (1.0 / _D)
    safe = jnp.where(seg > 0.0, seg, 1.0)
    aggr = (a - seg) / safe
    x = x_ref[...]

    hg = jnp.maximum(_dot(x, wg0a_ref[...]) + _dot(aggr, wg0b_ref[...])
                     + bg0_ref[...], 0.0)
    lg = _dot(hg, wg1_ref[...]) + bg1_ref[...]
    part = jnp.sum(lg, axis=0, keepdims=True)

    @pl.when(t == 0)
    def _():
        acc_ref[...] = part

    @pl.when(t > 0)
    def _():
        acc_ref[...] = acc_ref[...] + part

    hn = jnp.maximum(_dot(x, wn0a_ref[...]) + _dot(aggr, wn0b_ref[...])
                     + bn0_ref[...], 0.0)
    o = _dot(hn, wn1_ref[...]) + bn1_ref[...]
    mu = jnp.mean(o, axis=-1, keepdims=True)
    var = jnp.mean((o - mu) * (o - mu), axis=-1, keepdims=True)
    out_ref[...] = (o - mu) * lax.rsqrt(var + 1e-5) + x
    seg_ref[...] = seg

    @pl.when(t == (_N // _TN) - 1)
    def _():
        y = (acc_ref[...] * (1.0 / _N) + gn_ref[...]) * tinv_ref[...]
        mx = jnp.max(y, axis=-1, keepdims=True)
        e = jnp.exp(y - mx)
        ys = e / jnp.sum(e, axis=-1, keepdims=True)
        am = jnp.argmax(ys, axis=-1)[:, None]
        io = jax.lax.broadcasted_iota(jnp.int32, (1, 2), 1)
        oh = (io == am).astype(jnp.float32)
        yh_ref[...] = (oh - ys) + ys


def _tcc(x, a, wg0a, wg0b, bg0, wg1, bg1, wn0a, wn0b, bn0, wn1, bn1,
         gn, tinv):
    full = lambda a_, b_: pl.BlockSpec((a_, b_), lambda t: (0, 0))
    nt = _N // _TN
    return pl.pallas_call(
        _tcc_body,
        grid=(nt,),
        in_specs=[
            pl.BlockSpec((_TN, _D), lambda t: (t, 0)),
            pl.BlockSpec((_TN, _D), lambda t: (t, 0)),
            pl.BlockSpec((_TN, _D), lambda t: (t + nt, 0)),
            full(_D, _D), full(_D, _D), full(1, _D),
            full(_D, 2), full(1, 2),
            full(_D, _D), full(_D, _D), full(1, _D),
            full(_D, _D), full(1, _D),
            full(1, 2), full(1, 1),
        ],
        out_specs=[
            pl.BlockSpec((_TN, _D), lambda t: (t, 0)),
            pl.BlockSpec((_TN, 1), lambda t: (t, 0)),
            pl.BlockSpec((1, 2), lambda t: (0, 0)),
            pl.BlockSpec((1, 2), lambda t: (0, 0)),
        ],
        out_shape=[
            jax.ShapeDtypeStruct((_N, _D), jnp.float32),
            jax.ShapeDtypeStruct((_N, 1), jnp.float32),
            jax.ShapeDtypeStruct((1, 2), jnp.float32),
            jax.ShapeDtypeStruct((1, 2), jnp.float32),
        ],
    )(x, a, a, wg0a, wg0b, bg0, wg1, bg1, wn0a, wn0b, bn0, wn1, bn1,
      gn, tinv)


def _sc3_body(wexp_hbm, gj_hbm, seg_hbm, wn_hbm, segv, idx_j, wbuf):
    wid = lax.axis_index("s") * 2 + lax.axis_index("c")
    base = wid * _EPW
    pltpu.sync_copy(seg_hbm, segv)
    pltpu.sync_copy(gj_hbm.at[pl.ds(base, _EPW)], idx_j)
    pltpu.sync_copy(wexp_hbm.at[pl.ds(base, _EPW)], wbuf)

    def grp(k, c2):
        sl = pl.ds(k * 16, 16)
        sv = plsc.load_gather(segv, [idx_j[sl]])
        wbuf[sl] = wbuf[sl] / sv
        return c2

    lax.fori_loop(0, _EPW // 16, grp, 0)
    pltpu.sync_copy(wbuf, wn_hbm.at[pl.ds(base, _EPW)])


def _sc3(wexp, gj, seg):
    f = functools.partial(
        pl.kernel,
        mesh=_mesh(),
        compiler_params=_SC_PARAMS,
        out_type=[jax.ShapeDtypeStruct((_E,), jnp.float32)],
        scratch_types=[
            pltpu.VMEM((_N,), jnp.float32),
            pltpu.VMEM((_EPW,), jnp.int32),
            pltpu.VMEM((_EPW,), jnp.float32),
        ],
    )(_sc3_body)
    return f(wexp, gj, seg)


def kernel(x, g, pos, temp, W_ei0, b_ei0, W_ei1, b_ei1, W_ew0, b_ew0,
           W_ew1, b_ew1, W_g0, b_g0, W_g1, b_g1, W_nd0, b_nd0, W_nd1,
           b_nd1):
    gi = g[0].astype(jnp.int32)
    gj = g[1].astype(jnp.int32)
    gi2 = gi.reshape(_NW, _NCH, _C)
    gj2 = gj.reshape(_NW, _NCH, _C)
    p4 = jnp.pad(pos.astype(jnp.float32), ((0, 0), (0, 1)))
    pos3 = pos.astype(jnp.float32).reshape(3 * _N)

    wa = W_ei0[4:4 + _D]
    wb = W_ei0[4 + _D:4 + 2 * _D]
    wf4 = jnp.pad(W_ei0[0:3], ((0, 1), (0, 0)))
    wnrm = W_ei0[3:4]
    b0 = b_ei0.reshape(1, _D)
    b1 = b_ei1.reshape(1, _D)
    be0 = b_ew0.reshape(1, _D)
    we1row = W_ew1.reshape(1, _D)
    be1 = b_ew1.reshape(1, 1)
    wg0a = W_g0[:_D]
    wg0b = W_g0[_D:]
    bg0 = b_g0.reshape(1, _D)
    bg1 = b_g1.reshape(1, 2)
    wn0a = W_nd0[:_D]
    wn0b = W_nd0[_D:]
    bn0 = b_nd0.reshape(1, _D)
    bn1 = b_nd1.reshape(1, _D)

    u = jax.random.uniform(jax.random.key(42), (2,), minval=1e-10,
                           maxval=1.0)
    gn = (-jnp.log(-jnp.log(u))).reshape(1, 2)
    tinv = (1.0 / jnp.asarray(temp, jnp.float32)).reshape(1, 1)

    xa, xb = _tca(x, p4, wa, wb, wf4)
    s, nsq = _sc1(xa, xb, pos3, gi2, gj2)
    m, wexp = _tcb(s, nsq.reshape(_E, 1), wnrm, b0, W_ei1, b1, W_ew0,
                   be0, we1row, be1)
    (a,) = _sc2(m, gj2)
    out, seg, _acc, yh = _tcc(x, a, wg0a, wg0b, bg0, W_g1, bg1, wn0a,
                              wn0b, bn0, W_nd1, bn1, gn, tinv)
    (wn,) = _sc3(wexp.reshape(_E), gj, seg.reshape(_N))
    return out, wn.reshape(_E, 1), yh.reshape(2)
